# Initial kernel scaffold; baseline (speedup 1.0000x reference)
import jax
import jax.numpy as jnp
from jax.experimental import pallas as pl


def kernel(q, k):
    raise NotImplementedError("write your pallas kernel here")


# speedup vs baseline: 1.6013x; 1.6013x over previous
import math

import jax
import jax.numpy as jnp
from jax.experimental import pallas as pl

_SEL = 513
_SENTINEL = 50.0


def _z_kernel(q_ref, k_ref, g_ref, z_ref):
    kk = k_ref[0]
    a = jax.lax.dot_general(
        q_ref[0], kk, (((1,), (1,)), ((), ())),
        preferred_element_type=jnp.float32,
    )
    a = a / math.sqrt(kk.shape[-1])
    col = jax.lax.broadcasted_iota(jnp.int32, a.shape, 1)
    is0 = col == 0
    am = jnp.where(is0, -jnp.inf, a)
    m = jnp.max(am)
    e = jnp.where(is0, 0.0, jnp.exp(am - m))
    p = e / jnp.sum(e)
    z = jnp.log(p + 1e-20) + g_ref[0]
    z_ref[0] = jnp.where(is0, _SENTINEL, z)


def _select_kernel(z_ref, o_ref):
    z = z_ref[...]
    s = jax.lax.bitcast_convert_type(z, jnp.int32)
    key = jnp.where(s < 0, jnp.int32(-1) - (s & jnp.int32(0x7FFFFFFF)), s)
    col = jax.lax.broadcasted_iota(jnp.int32, key.shape, 1)

    def count_ge(t):
        return jnp.sum((key >= t).astype(jnp.int32), axis=1, keepdims=True)

    lo = jnp.min(key, axis=1, keepdims=True)
    hi = jnp.max(key, axis=1, keepdims=True)

    def body(_, lohi):
        lo, hi = lohi
        mid = (lo >> 1) + (hi >> 1) + (lo & hi & 1)
        pred = count_ge(mid) >= _SEL
        return jnp.where(pred, mid, lo), jnp.where(pred, hi, mid)

    lo, hi = jax.lax.fori_loop(0, 32, body, (lo, hi))
    v = lo
    cnt_gt = count_ge(v + 1)
    need = _SEL - cnt_gt
    tie = (key == v).astype(jnp.int32)

    def body2(_, clochi):
        clo, chi = clochi
        mid = (clo + chi) >> 1
        cnt = jnp.sum(jnp.where(col <= mid, tie, 0), axis=1, keepdims=True)
        pred = cnt >= need
        return jnp.where(pred, clo, mid), jnp.where(pred, mid, chi)

    S = key.shape[1]
    clo = jnp.full_like(v, -1)
    chi = jnp.full_like(v, S - 1)
    _, cut = jax.lax.fori_loop(0, 14, body2, (clo, chi))
    mask = (key > v) | ((key == v) & (col <= cut))
    o_ref[...] = mask.astype(jnp.int32)


def kernel(q, k):
    B, S, D = q.shape
    g = jax.random.gumbel(jax.random.key(42), (B, S - 1), dtype=jnp.float32)
    gp = jnp.pad(g, ((0, 0), (1, 0)))
    gp = gp.reshape(B, 1, S)
    q0 = q[:, :1, :]
    z = pl.pallas_call(
        _z_kernel,
        grid=(B,),
        in_specs=[
            pl.BlockSpec((1, 1, D), lambda b: (b, 0, 0)),
            pl.BlockSpec((1, S, D), lambda b: (b, 0, 0)),
            pl.BlockSpec((1, 1, S), lambda b: (b, 0, 0)),
        ],
        out_specs=pl.BlockSpec((1, 1, S), lambda b: (b, 0, 0)),
        out_shape=jax.ShapeDtypeStruct((B, 1, S), jnp.float32),
    )(q0, k, gp)
    z = z.reshape(B, S)
    m = pl.pallas_call(
        _select_kernel,
        out_shape=jax.ShapeDtypeStruct((B, S), jnp.int32),
    )(z)
    return m.astype(bool)
